# baseline (speedup 1.0000x reference)
import jax
import jax.numpy as jnp
from jax.experimental import pallas as pl
from jax.experimental.pallas import tpu as pltpu

B, T, NMEL = 4, 600, 80
STRIDE = 4
IN_DIM = NMEL * STRIDE
D_MODEL = 512
NCB = 4
EMB_DIM = 16
NUM_EMB = 8192
N = T // STRIDE
ROWS = B * N
EC = 2048
NE = NUM_EMB // EC
RB = 200
NRB = ROWS // RB
NEG = -1e30


def _ln(x, eps=1e-6):
    m = jnp.mean(x, axis=-1, keepdims=True)
    s = x - m
    v = jnp.mean(s * s, axis=-1, keepdims=True)
    return s / jnp.sqrt(v + eps)


def _prep_kernel(x_ref, proj_ref, w_in_ref, b_in_ref,
                 w_ff1_ref, b_ff1_ref, w_ff2_ref, b_ff2_ref,
                 enc_ref, xsn_ref):
    x = x_ref[...]
    y = _ln(x)
    xs = jnp.dot(y, proj_ref[...], preferred_element_type=jnp.float32)
    nrm = jnp.sqrt(jnp.sum(xs * xs, axis=-1, keepdims=True))
    xsn_ref[...] = xs / (nrm + 1e-8)
    h1 = jnp.dot(x, w_in_ref[...], preferred_element_type=jnp.float32) \
        + b_in_ref[...]
    t = _ln(h1)
    f = jax.nn.gelu(
        jnp.dot(t, w_ff1_ref[...], preferred_element_type=jnp.float32)
        + b_ff1_ref[...])
    h2 = h1 + jnp.dot(f, w_ff2_ref[...], preferred_element_type=jnp.float32) \
        + b_ff2_ref[...]
    enc_ref[...] = _ln(h2)


def _codes_kernel(xsn_ref, embt_ref, codes_ref, runmin_ref, runidx_ref):
    ec = pl.program_id(1)

    @pl.when(ec == 0)
    def _():
        runmin_ref[...] = jnp.full_like(runmin_ref, jnp.inf)
        runidx_ref[...] = jnp.full_like(runidx_ref, NUM_EMB)

    xq = xsn_ref[0]
    HC = EC // 2
    iota = jax.lax.broadcasted_iota(jnp.int32, (ROWS, HC), 1)
    for h in range(2):
        et = embt_ref[0, :, h * HC:(h + 1) * HC]
        c2 = jnp.sum(et * et, axis=0, keepdims=True)
        d = c2 - 2.0 * jnp.dot(xq, et, preferred_element_type=jnp.float32)
        m = jnp.min(d, axis=1, keepdims=True)
        idx = jnp.min(jnp.where(d == m, iota, NUM_EMB), axis=1,
                      keepdims=True) + (ec * EC + h * HC)
        upd = m < runmin_ref[...]
        runidx_ref[...] = jnp.where(upd, idx, runidx_ref[...])
        runmin_ref[...] = jnp.where(upd, m, runmin_ref[...])

    @pl.when(ec == NE - 1)
    def _():
        codes_ref[0] = runidx_ref[...]


def _loss_kernel(enc_ref, w_ref, tgt_ref, vals_ref, valid_ref,
                 nll_ref, corr_ref, uniq_ref,
                 runm_ref, runs_ref, tl_ref, pres_ref):
    cb = pl.program_id(0)
    ec = pl.program_id(1)
    r = pl.program_id(2)
    rows = pl.ds(r * RB, RB)

    @pl.when(jnp.logical_and(jnp.logical_and(cb == 0, ec == 0), r == 0))
    def _():
        nll_ref[...] = jnp.zeros_like(nll_ref)
        corr_ref[...] = jnp.zeros_like(corr_ref)
        uniq_ref[...] = jnp.zeros_like(uniq_ref)
        pres_ref[...] = jnp.zeros_like(pres_ref)

    @pl.when(ec == 0)
    def _():
        runm_ref[rows, :] = jnp.full((RB, 1), NEG, jnp.float32)
        runs_ref[rows, :] = jnp.zeros((RB, 1), jnp.float32)
        tl_ref[rows, :] = jnp.zeros((RB, 1), jnp.float32)

    HC = EC // 2
    enc = enc_ref[...]
    L0 = jnp.dot(enc, w_ref[0, :, :HC], preferred_element_type=jnp.float32)
    L1 = jnp.dot(enc, w_ref[0, :, HC:], preferred_element_type=jnp.float32)
    iota = jax.lax.broadcasted_iota(jnp.int32, (RB, HC), 1)
    off = ec * EC
    tgt0 = tgt_ref[0] - off
    tgt1 = tgt0 - HC
    vals0 = vals_ref[0] - off
    vals1 = vals0 - HC

    cm = jnp.maximum(jnp.max(L0, axis=1, keepdims=True),
                     jnp.max(L1, axis=1, keepdims=True))
    newm = jnp.maximum(runm_ref[rows, :], cm)
    runs_ref[rows, :] = runs_ref[rows, :] * jnp.exp(runm_ref[rows, :] - newm) \
        + jnp.sum(jnp.exp(L0 - newm), axis=1, keepdims=True) \
        + jnp.sum(jnp.exp(L1 - newm), axis=1, keepdims=True)
    runm_ref[rows, :] = newm

    tl_ref[rows, :] = tl_ref[rows, :] \
        + jnp.sum(jnp.where(iota == tgt0, L0, 0.0), axis=1, keepdims=True) \
        + jnp.sum(jnp.where(iota == tgt1, L1, 0.0), axis=1, keepdims=True)

    pres_ref[pl.ds(2 * ec, 1), :] = pres_ref[pl.ds(2 * ec, 1), :] + jnp.sum(
        (iota == vals0).astype(jnp.float32), axis=0, keepdims=True)
    pres_ref[pl.ds(2 * ec + 1, 1), :] = pres_ref[pl.ds(2 * ec + 1, 1), :] \
        + jnp.sum((iota == vals1).astype(jnp.float32), axis=0, keepdims=True)

    @pl.when(ec == NE - 1)
    def _():
        v = valid_ref[...]
        lse = runm_ref[rows, :] + jnp.log(runs_ref[rows, :])
        nll_ref[...] = nll_ref[...] + jnp.sum(v * (lse - tl_ref[rows, :]))
        corr_ref[...] = corr_ref[...] + jnp.sum(
            v * (tl_ref[rows, :] == runm_ref[rows, :]).astype(jnp.float32))

    last = (cb == NCB - 1) & (ec == NE - 1) & (r == NRB - 1)

    @pl.when(last)
    def _():
        uniq_ref[...] = jnp.zeros_like(uniq_ref) + jnp.sum(
            (pres_ref[...] > 0).astype(jnp.float32))


def kernel(feats, feats_lengths, projection, embeddings, W_in, b_in,
           W_ff1, b_ff1, W_ff2, b_ff2, top_n_out):
    x = feats.reshape(ROWS, IN_DIM)
    embT = jnp.transpose(embeddings, (1, 2, 0))

    enc, xsn = pl.pallas_call(
        _prep_kernel,
        out_shape=[jax.ShapeDtypeStruct((ROWS, D_MODEL), jnp.float32),
                   jax.ShapeDtypeStruct((ROWS, NCB * EMB_DIM), jnp.float32)],
    )(x, projection, W_in, b_in.reshape(1, -1),
      W_ff1, b_ff1.reshape(1, -1), W_ff2, b_ff2.reshape(1, -1))

    xsn3 = jnp.transpose(xsn.reshape(ROWS, NCB, EMB_DIM), (1, 0, 2))

    codes = pl.pallas_call(
        _codes_kernel,
        grid=(NCB, NE),
        in_specs=[
            pl.BlockSpec((1, ROWS, EMB_DIM), lambda cb, ec: (cb, 0, 0)),
            pl.BlockSpec((1, EMB_DIM, EC), lambda cb, ec: (cb, 0, ec)),
        ],
        out_specs=pl.BlockSpec((1, ROWS, 1), lambda cb, ec: (cb, 0, 0)),
        out_shape=jax.ShapeDtypeStruct((NCB, ROWS, 1), jnp.int32),
        scratch_shapes=[pltpu.VMEM((ROWS, 1), jnp.float32),
                        pltpu.VMEM((ROWS, 1), jnp.int32)],
    )(xsn3, embT)

    lim = feats_lengths // STRIDE
    t_idx = jnp.arange(N)
    validf = ((t_idx[None, :] + 1) < lim[:, None]).astype(jnp.float32) \
        .reshape(ROWS, 1)
    codes2 = codes[:, :, 0]
    tgt = jnp.roll(codes2, -1, axis=1).reshape(NCB, ROWS, 1)
    tmask = (t_idx[None, :] < lim[:, None]).reshape(1, ROWS)
    t0 = (jnp.arange(ROWS) % N == 0)[None, :]
    vals2 = jnp.where(t0, -1, jnp.where(tmask, codes2, 0)) \
        .reshape(NCB, ROWS, 1).astype(jnp.int32)
    mask_sum = jnp.sum(jnp.maximum(lim - 1, 0).astype(jnp.float32))

    nll, corr, uniq = pl.pallas_call(
        _loss_kernel,
        grid=(NCB, NE, NRB),
        in_specs=[
            pl.BlockSpec((RB, D_MODEL), lambda cb, ec, r: (r, 0)),
            pl.BlockSpec((1, D_MODEL, EC), lambda cb, ec, r: (cb, 0, ec)),
            pl.BlockSpec((1, RB, 1), lambda cb, ec, r: (cb, r, 0)),
            pl.BlockSpec((1, RB, 1), lambda cb, ec, r: (cb, r, 0)),
            pl.BlockSpec((RB, 1), lambda cb, ec, r: (r, 0)),
        ],
        out_specs=[pl.BlockSpec((1, 1), lambda cb, ec, r: (0, 0))] * 3,
        out_shape=[jax.ShapeDtypeStruct((1, 1), jnp.float32)] * 3,
        scratch_shapes=[pltpu.VMEM((ROWS, 1), jnp.float32),
                        pltpu.VMEM((ROWS, 1), jnp.float32),
                        pltpu.VMEM((ROWS, 1), jnp.float32),
                        pltpu.VMEM((2 * NE, EC // 2), jnp.float32)],
    )(enc, top_n_out[0], tgt, vals2, validf)

    num_codes = mask_sum * NCB
    loss = nll[0, 0] / num_codes
    codes_acc = corr[0, 0] / num_codes
    return (codes_acc, loss, num_codes, uniq[0, 0].astype(jnp.int32))


# speedup vs baseline: 1.1478x; 1.1478x over previous
import jax
import jax.numpy as jnp
from jax.experimental import pallas as pl
from jax.experimental.pallas import tpu as pltpu

B, T, NMEL = 4, 600, 80
STRIDE = 4
IN_DIM = NMEL * STRIDE
D_MODEL = 512
NCB = 4
EMB_DIM = 16
NUM_EMB = 8192
N = T // STRIDE
ROWS = B * N
EC = 2048
NE = NUM_EMB // EC
NE2 = NE // 2
RB = 200
NRB = ROWS // RB
NEG = -1e30


def _ln(x, eps=1e-6):
    m = jnp.mean(x, axis=-1, keepdims=True)
    s = x - m
    v = jnp.mean(s * s, axis=-1, keepdims=True)
    return s / jnp.sqrt(v + eps)


def _prep_kernel(x_ref, proj_ref, w_in_ref, b_in_ref,
                 w_ff1_ref, b_ff1_ref, w_ff2_ref, b_ff2_ref,
                 enc_ref, xsn_ref):
    x = x_ref[...]
    y = _ln(x)
    xs = jnp.dot(y, proj_ref[...], preferred_element_type=jnp.float32)
    nrm = jnp.sqrt(jnp.sum(xs * xs, axis=-1, keepdims=True))
    xsn_ref[...] = xs / (nrm + 1e-8)
    h1 = jnp.dot(x, w_in_ref[...], preferred_element_type=jnp.float32) \
        + b_in_ref[...]
    t = _ln(h1)
    f = jax.nn.gelu(
        jnp.dot(t, w_ff1_ref[...], preferred_element_type=jnp.float32)
        + b_ff1_ref[...])
    h2 = h1 + jnp.dot(f, w_ff2_ref[...], preferred_element_type=jnp.float32) \
        + b_ff2_ref[...]
    enc_ref[...] = _ln(h2)


def _codes_kernel(xsn_ref, embt_ref, codes_ref, runmin_ref, runidx_ref):
    ec = pl.program_id(1)

    @pl.when(ec == 0)
    def _():
        runmin_ref[...] = jnp.full_like(runmin_ref, jnp.inf)
        runidx_ref[...] = jnp.full_like(runidx_ref, NUM_EMB)

    xq = xsn_ref[0]
    HC = EC // 2
    iota = jax.lax.broadcasted_iota(jnp.int32, (ROWS, HC), 1)
    for h in range(2):
        et = embt_ref[0, :, h * HC:(h + 1) * HC]
        c2 = jnp.sum(et * et, axis=0, keepdims=True)
        d = c2 - 2.0 * jnp.dot(xq, et, preferred_element_type=jnp.float32)
        m = jnp.min(d, axis=1, keepdims=True)
        idx = jnp.min(jnp.where(d == m, iota, NUM_EMB), axis=1,
                      keepdims=True) + (ec * EC + h * HC)
        upd = m < runmin_ref[...]
        runidx_ref[...] = jnp.where(upd, idx, runidx_ref[...])
        runmin_ref[...] = jnp.where(upd, m, runmin_ref[...])

    @pl.when(ec == NE - 1)
    def _():
        codes_ref[0] = runidx_ref[...]


def _loss_kernel(enc_ref, w0_ref, w1_ref, tgt_ref, vals_ref, valid_ref,
                 nll_ref, corr_ref, uniq_ref,
                 runm_ref, runs_ref, tl_ref, pres_ref):
    cb = pl.program_id(0)
    ec = pl.program_id(1)
    r = pl.program_id(2)
    rows = pl.ds(r * RB, RB)

    @pl.when(jnp.logical_and(jnp.logical_and(cb == 0, ec == 0), r == 0))
    def _():
        nll_ref[...] = jnp.zeros_like(nll_ref)
        corr_ref[...] = jnp.zeros_like(corr_ref)
        uniq_ref[...] = jnp.zeros_like(uniq_ref)
        pres_ref[...] = jnp.zeros_like(pres_ref)

    @pl.when(ec == 0)
    def _():
        runm_ref[rows, :] = jnp.full((RB, 1), NEG, jnp.float32)
        runs_ref[rows, :] = jnp.zeros((RB, 1), jnp.float32)
        tl_ref[rows, :] = jnp.zeros((RB, 1), jnp.float32)

    enc = enc_ref[...]
    L0 = jnp.dot(enc, w0_ref[0], preferred_element_type=jnp.float32)
    L1 = jnp.dot(enc, w1_ref[0], preferred_element_type=jnp.float32)
    iota = jax.lax.broadcasted_iota(jnp.int32, (RB, EC), 1)
    off = ec * (2 * EC)
    tgt0 = tgt_ref[0] - off
    tgt1 = tgt0 - EC
    vals0 = vals_ref[0] - off
    vals1 = vals0 - EC

    cm = jnp.maximum(jnp.max(L0, axis=1, keepdims=True),
                     jnp.max(L1, axis=1, keepdims=True))
    newm = jnp.maximum(runm_ref[rows, :], cm)
    runs_ref[rows, :] = runs_ref[rows, :] * jnp.exp(runm_ref[rows, :] - newm) \
        + jnp.sum(jnp.exp(L0 - newm), axis=1, keepdims=True) \
        + jnp.sum(jnp.exp(L1 - newm), axis=1, keepdims=True)
    runm_ref[rows, :] = newm

    tl_ref[rows, :] = tl_ref[rows, :] \
        + jnp.sum(jnp.where(iota == tgt0, L0, 0.0), axis=1, keepdims=True) \
        + jnp.sum(jnp.where(iota == tgt1, L1, 0.0), axis=1, keepdims=True)

    pres_ref[pl.ds(2 * ec, 1), :] = pres_ref[pl.ds(2 * ec, 1), :] + jnp.sum(
        (iota == vals0).astype(jnp.float32), axis=0, keepdims=True)
    pres_ref[pl.ds(2 * ec + 1, 1), :] = pres_ref[pl.ds(2 * ec + 1, 1), :] \
        + jnp.sum((iota == vals1).astype(jnp.float32), axis=0, keepdims=True)

    @pl.when(ec == NE2 - 1)
    def _():
        v = valid_ref[...]
        lse = runm_ref[rows, :] + jnp.log(runs_ref[rows, :])
        nll_ref[...] = nll_ref[...] + jnp.sum(v * (lse - tl_ref[rows, :]))
        corr_ref[...] = corr_ref[...] + jnp.sum(
            v * (tl_ref[rows, :] == runm_ref[rows, :]).astype(jnp.float32))

    last = (cb == NCB - 1) & (ec == NE2 - 1) & (r == NRB - 1)

    @pl.when(last)
    def _():
        uniq_ref[...] = jnp.zeros_like(uniq_ref) + jnp.sum(
            (pres_ref[...] > 0).astype(jnp.float32))


def kernel(feats, feats_lengths, projection, embeddings, W_in, b_in,
           W_ff1, b_ff1, W_ff2, b_ff2, top_n_out):
    x = feats.reshape(ROWS, IN_DIM)
    embT = jnp.transpose(embeddings, (1, 2, 0))

    enc, xsn = pl.pallas_call(
        _prep_kernel,
        out_shape=[jax.ShapeDtypeStruct((ROWS, D_MODEL), jnp.float32),
                   jax.ShapeDtypeStruct((ROWS, NCB * EMB_DIM), jnp.float32)],
    )(x, projection, W_in, b_in.reshape(1, -1),
      W_ff1, b_ff1.reshape(1, -1), W_ff2, b_ff2.reshape(1, -1))

    xsn3 = jnp.transpose(xsn.reshape(ROWS, NCB, EMB_DIM), (1, 0, 2))

    codes = pl.pallas_call(
        _codes_kernel,
        grid=(NCB, NE),
        in_specs=[
            pl.BlockSpec((1, ROWS, EMB_DIM), lambda cb, ec: (cb, 0, 0)),
            pl.BlockSpec((1, EMB_DIM, EC), lambda cb, ec: (cb, 0, ec)),
        ],
        out_specs=pl.BlockSpec((1, ROWS, 1), lambda cb, ec: (cb, 0, 0)),
        out_shape=jax.ShapeDtypeStruct((NCB, ROWS, 1), jnp.int32),
        scratch_shapes=[pltpu.VMEM((ROWS, 1), jnp.float32),
                        pltpu.VMEM((ROWS, 1), jnp.int32)],
    )(xsn3, embT)

    lim = feats_lengths // STRIDE
    t_idx = jnp.arange(N)
    validf = ((t_idx[None, :] + 1) < lim[:, None]).astype(jnp.float32) \
        .reshape(ROWS, 1)
    codes2 = codes[:, :, 0]
    tgt = jnp.roll(codes2, -1, axis=1).reshape(NCB, ROWS, 1)
    tmask = (t_idx[None, :] < lim[:, None]).reshape(1, ROWS)
    t0 = (jnp.arange(ROWS) % N == 0)[None, :]
    vals2 = jnp.where(t0, -1, jnp.where(tmask, codes2, 0)) \
        .reshape(NCB, ROWS, 1).astype(jnp.int32)
    mask_sum = jnp.sum(jnp.maximum(lim - 1, 0).astype(jnp.float32))

    W = top_n_out[0]
    nll, corr, uniq = pl.pallas_call(
        _loss_kernel,
        grid=(NCB, NE2, NRB),
        in_specs=[
            pl.BlockSpec((RB, D_MODEL), lambda cb, ec, r: (r, 0)),
            pl.BlockSpec((1, D_MODEL, EC), lambda cb, ec, r: (cb, 0, 2 * ec)),
            pl.BlockSpec((1, D_MODEL, EC),
                         lambda cb, ec, r: (cb, 0, 2 * ec + 1)),
            pl.BlockSpec((1, RB, 1), lambda cb, ec, r: (cb, r, 0)),
            pl.BlockSpec((1, RB, 1), lambda cb, ec, r: (cb, r, 0)),
            pl.BlockSpec((RB, 1), lambda cb, ec, r: (r, 0)),
        ],
        out_specs=[pl.BlockSpec((1, 1), lambda cb, ec, r: (0, 0))] * 3,
        out_shape=[jax.ShapeDtypeStruct((1, 1), jnp.float32)] * 3,
        scratch_shapes=[pltpu.VMEM((ROWS, 1), jnp.float32),
                        pltpu.VMEM((ROWS, 1), jnp.float32),
                        pltpu.VMEM((ROWS, 1), jnp.float32),
                        pltpu.VMEM((NE, EC), jnp.float32)],
    )(enc, W, W, tgt, vals2, validf)

    num_codes = mask_sum * NCB
    loss = nll[0, 0] / num_codes
    codes_acc = corr[0, 0] / num_codes
    return (codes_acc, loss, num_codes, uniq[0, 0].astype(jnp.int32))


# speedup vs baseline: 1.1846x; 1.0321x over previous
import jax
import jax.numpy as jnp
from jax.experimental import pallas as pl
from jax.experimental.pallas import tpu as pltpu

B, T, NMEL = 4, 600, 80
STRIDE = 4
IN_DIM = NMEL * STRIDE
D_MODEL = 512
NCB = 4
EMB_DIM = 16
NUM_EMB = 8192
N = T // STRIDE
ROWS = B * N
EC = 2048
NE = NUM_EMB // EC
NE2 = NE // 2
RB = 200
NRB = ROWS // RB
NEG = -1e30


def _ln(x, eps=1e-6):
    m = jnp.mean(x, axis=-1, keepdims=True)
    s = x - m
    v = jnp.mean(s * s, axis=-1, keepdims=True)
    return s / jnp.sqrt(v + eps)


def _prep_kernel(x_ref, proj_ref, w_in_ref, b_in_ref,
                 w_ff1_ref, b_ff1_ref, w_ff2_ref, b_ff2_ref,
                 enc_ref, xsn_ref):
    x = x_ref[...]
    y = _ln(x)
    xs = jnp.dot(y, proj_ref[...], preferred_element_type=jnp.float32)
    nrm = jnp.sqrt(jnp.sum(xs * xs, axis=-1, keepdims=True))
    xsn_ref[...] = xs / (nrm + 1e-8)
    h1 = jnp.dot(x, w_in_ref[...], preferred_element_type=jnp.float32) \
        + b_in_ref[...]
    t = _ln(h1)
    f = jax.nn.gelu(
        jnp.dot(t, w_ff1_ref[...], preferred_element_type=jnp.float32)
        + b_ff1_ref[...])
    h2 = h1 + jnp.dot(f, w_ff2_ref[...], preferred_element_type=jnp.float32) \
        + b_ff2_ref[...]
    enc_ref[...] = _ln(h2)


def _codes_kernel(xsn_ref, embt_ref, codes_ref, runmin_ref, runidx_ref):
    ec = pl.program_id(1)

    @pl.when(ec == 0)
    def _():
        runmin_ref[...] = jnp.full_like(runmin_ref, jnp.inf)
        runidx_ref[...] = jnp.full_like(runidx_ref, NUM_EMB)

    xq = xsn_ref[0]
    HC = EC // 2
    iota = jax.lax.broadcasted_iota(jnp.int32, (ROWS, HC), 1)
    for h in range(2):
        et = embt_ref[0, :, h * HC:(h + 1) * HC]
        c2 = jnp.sum(et * et, axis=0, keepdims=True)
        d = c2 - 2.0 * jnp.dot(xq, et, preferred_element_type=jnp.float32)
        m = jnp.min(d, axis=1, keepdims=True)
        idx = jnp.min(jnp.where(d == m, iota, NUM_EMB), axis=1,
                      keepdims=True) + (ec * EC + h * HC)
        upd = m < runmin_ref[...]
        runidx_ref[...] = jnp.where(upd, idx, runidx_ref[...])
        runmin_ref[...] = jnp.where(upd, m, runmin_ref[...])

    @pl.when(ec == NE - 1)
    def _():
        codes_ref[0] = runidx_ref[...]


def _loss_kernel(enc_ref, w0_ref, w1_ref, w2_ref, w3_ref,
                 tgt_ref, vals_ref, valid_ref,
                 nll_ref, corr_ref, uniq_ref, pres_ref):
    cb = pl.program_id(0)
    r = pl.program_id(1)

    @pl.when(jnp.logical_and(cb == 0, r == 0))
    def _():
        nll_ref[...] = jnp.zeros_like(nll_ref)
        corr_ref[...] = jnp.zeros_like(corr_ref)
        uniq_ref[...] = jnp.zeros_like(uniq_ref)
        pres_ref[...] = jnp.zeros_like(pres_ref)

    enc = enc_ref[...]
    Ls = [jnp.dot(enc, w[0], preferred_element_type=jnp.float32)
          for w in (w0_ref, w1_ref, w2_ref, w3_ref)]
    iota = jax.lax.broadcasted_iota(jnp.int32, (RB, EC), 1)
    tgt = tgt_ref[0]
    vals = vals_ref[0]
    v = valid_ref[...]

    m = jnp.max(Ls[0], axis=1, keepdims=True)
    for L in Ls[1:]:
        m = jnp.maximum(m, jnp.max(L, axis=1, keepdims=True))
    se = jnp.sum(jnp.exp(Ls[0] - m), axis=1, keepdims=True)
    for L in Ls[1:]:
        se = se + jnp.sum(jnp.exp(L - m), axis=1, keepdims=True)
    lse = m + jnp.log(se)

    tl = jnp.sum(jnp.where(iota == tgt, Ls[0], 0.0), axis=1, keepdims=True)
    for h, L in enumerate(Ls[1:], start=1):
        tl = tl + jnp.sum(jnp.where(iota == (tgt - h * EC), L, 0.0),
                          axis=1, keepdims=True)

    nll_ref[...] = nll_ref[...] + jnp.sum(v * (lse - tl))
    corr_ref[...] = corr_ref[...] + jnp.sum(
        v * (tl == m).astype(jnp.float32))

    for h, L in enumerate(Ls):
        pres_ref[h:h + 1, :] = pres_ref[h:h + 1, :] + jnp.sum(
            (iota == (vals - h * EC)).astype(jnp.float32),
            axis=0, keepdims=True)

    @pl.when(jnp.logical_and(cb == NCB - 1, r == NRB - 1))
    def _():
        uniq_ref[...] = jnp.zeros_like(uniq_ref) + jnp.sum(
            (pres_ref[...] > 0).astype(jnp.float32))


def kernel(feats, feats_lengths, projection, embeddings, W_in, b_in,
           W_ff1, b_ff1, W_ff2, b_ff2, top_n_out):
    x = feats.reshape(ROWS, IN_DIM)
    embT = jnp.transpose(embeddings, (1, 2, 0))

    enc, xsn = pl.pallas_call(
        _prep_kernel,
        out_shape=[jax.ShapeDtypeStruct((ROWS, D_MODEL), jnp.float32),
                   jax.ShapeDtypeStruct((ROWS, NCB * EMB_DIM), jnp.float32)],
    )(x, projection, W_in, b_in.reshape(1, -1),
      W_ff1, b_ff1.reshape(1, -1), W_ff2, b_ff2.reshape(1, -1))

    xsn3 = jnp.transpose(xsn.reshape(ROWS, NCB, EMB_DIM), (1, 0, 2))

    codes = pl.pallas_call(
        _codes_kernel,
        grid=(NCB, NE),
        in_specs=[
            pl.BlockSpec((1, ROWS, EMB_DIM), lambda cb, ec: (cb, 0, 0)),
            pl.BlockSpec((1, EMB_DIM, EC), lambda cb, ec: (cb, 0, ec)),
        ],
        out_specs=pl.BlockSpec((1, ROWS, 1), lambda cb, ec: (cb, 0, 0)),
        out_shape=jax.ShapeDtypeStruct((NCB, ROWS, 1), jnp.int32),
        scratch_shapes=[pltpu.VMEM((ROWS, 1), jnp.float32),
                        pltpu.VMEM((ROWS, 1), jnp.int32)],
    )(xsn3, embT)

    lim = feats_lengths // STRIDE
    t_idx = jnp.arange(N)
    validf = ((t_idx[None, :] + 1) < lim[:, None]).astype(jnp.float32) \
        .reshape(ROWS, 1)
    codes2 = codes[:, :, 0]
    tgt = jnp.roll(codes2, -1, axis=1).reshape(NCB, ROWS, 1)
    tmask = (t_idx[None, :] < lim[:, None]).reshape(1, ROWS)
    t0 = (jnp.arange(ROWS) % N == 0)[None, :]
    vals2 = jnp.where(t0, -1, jnp.where(tmask, codes2, 0)) \
        .reshape(NCB, ROWS, 1).astype(jnp.int32)
    mask_sum = jnp.sum(jnp.maximum(lim - 1, 0).astype(jnp.float32))

    W = top_n_out[0]

    def _wspec(h):
        return pl.BlockSpec((1, D_MODEL, EC), lambda cb, r: (cb, 0, h))

    nll, corr, uniq = pl.pallas_call(
        _loss_kernel,
        grid=(NCB, NRB),
        in_specs=[
            pl.BlockSpec((RB, D_MODEL), lambda cb, r: (r, 0)),
            _wspec(0), _wspec(1), _wspec(2), _wspec(3),
            pl.BlockSpec((1, RB, 1), lambda cb, r: (cb, r, 0)),
            pl.BlockSpec((1, RB, 1), lambda cb, r: (cb, r, 0)),
            pl.BlockSpec((RB, 1), lambda cb, r: (r, 0)),
        ],
        out_specs=[pl.BlockSpec((1, 1), lambda cb, r: (0, 0))] * 3,
        out_shape=[jax.ShapeDtypeStruct((1, 1), jnp.float32)] * 3,
        scratch_shapes=[pltpu.VMEM((NE, EC), jnp.float32)],
    )(enc, W, W, W, W, tgt, vals2, validf)

    num_codes = mask_sum * NCB
    loss = nll[0, 0] / num_codes
    codes_acc = corr[0, 0] / num_codes
    return (codes_acc, loss, num_codes, uniq[0, 0].astype(jnp.int32))


# speedup vs baseline: 1.2586x; 1.0625x over previous
import jax
import jax.numpy as jnp
from jax.experimental import pallas as pl
from jax.experimental.pallas import tpu as pltpu

B, T, NMEL = 4, 600, 80
STRIDE = 4
IN_DIM = NMEL * STRIDE
D_MODEL = 512
NCB = 4
EMB_DIM = 16
NUM_EMB = 8192
N = T // STRIDE
ROWS = B * N
EC = 2048
NE = NUM_EMB // EC
NE2 = NE // 2
RB = 200
NRB = ROWS // RB
NEG = -1e30


def _ln(x, eps=1e-6):
    m = jnp.mean(x, axis=-1, keepdims=True)
    s = x - m
    v = jnp.mean(s * s, axis=-1, keepdims=True)
    return s / jnp.sqrt(v + eps)


def _prep_kernel(x_ref, proj_ref, w_in_ref, b_in_ref,
                 w_ff1a_ref, w_ff1b_ref, b_ff1_ref,
                 w_ff2a_ref, w_ff2b_ref, b_ff2_ref,
                 enc_ref, xsn_ref):
    x = x_ref[...]
    y = _ln(x)
    xs = jnp.dot(y, proj_ref[...], preferred_element_type=jnp.float32)
    nrm = jnp.sqrt(jnp.sum(xs * xs, axis=-1, keepdims=True))
    xsn_ref[...] = xs / (nrm + 1e-8)
    h1 = jnp.dot(x, w_in_ref[...], preferred_element_type=jnp.float32) \
        + b_in_ref[...]
    t = _ln(h1)
    FH = 2 * D_MODEL
    b_ff1 = b_ff1_ref[...]
    f0 = jax.nn.gelu(
        jnp.dot(t, w_ff1a_ref[...], preferred_element_type=jnp.float32)
        + b_ff1[:, :FH])
    f1 = jax.nn.gelu(
        jnp.dot(t, w_ff1b_ref[...], preferred_element_type=jnp.float32)
        + b_ff1[:, FH:])
    h2 = h1 \
        + jnp.dot(f0, w_ff2a_ref[...], preferred_element_type=jnp.float32) \
        + jnp.dot(f1, w_ff2b_ref[...], preferred_element_type=jnp.float32) \
        + b_ff2_ref[...]
    enc_ref[...] = _ln(h2)


def _codes_kernel(xsn_ref, embt_ref, tmask_ref, t0_ref,
                  tgt_ref, vals_ref, runmin_ref, runidx_ref):
    ec = pl.program_id(1)

    @pl.when(ec == 0)
    def _():
        runmin_ref[...] = jnp.full_like(runmin_ref, jnp.inf)
        runidx_ref[...] = jnp.full_like(runidx_ref, NUM_EMB)

    xq = xsn_ref[0]
    HC = EC // 2
    iota = jax.lax.broadcasted_iota(jnp.int32, (ROWS, HC), 1)
    for h in range(2):
        et = embt_ref[0, :, h * HC:(h + 1) * HC]
        c2 = jnp.sum(et * et, axis=0, keepdims=True)
        d = c2 - 2.0 * jnp.dot(xq, et, preferred_element_type=jnp.float32)
        m = jnp.min(d, axis=1, keepdims=True)
        idx = jnp.min(jnp.where(d == m, iota, NUM_EMB), axis=1,
                      keepdims=True) + (ec * EC + h * HC)
        upd = m < runmin_ref[...]
        runidx_ref[...] = jnp.where(upd, idx, runidx_ref[...])
        runmin_ref[...] = jnp.where(upd, m, runmin_ref[...])

    @pl.when(ec == NE - 1)
    def _():
        codes = runidx_ref[...]
        tgt_ref[0] = jnp.concatenate([codes[1:, :], codes[:1, :]], axis=0)
        vals_ref[0] = jnp.where(t0_ref[...] != 0, -1,
                                jnp.where(tmask_ref[...] != 0, codes, 0))


def _loss_kernel(enc_ref, w0_ref, w1_ref, w2_ref, w3_ref,
                 tgt_ref, vals_ref, valid_ref, msum_ref,
                 nll_ref, corr_ref, uniq_ref, pres_ref):
    cb = pl.program_id(0)
    r = pl.program_id(1)

    @pl.when(jnp.logical_and(cb == 0, r == 0))
    def _():
        nll_ref[...] = jnp.zeros_like(nll_ref)
        corr_ref[...] = jnp.zeros_like(corr_ref)
        uniq_ref[...] = jnp.zeros_like(uniq_ref)
        pres_ref[...] = jnp.zeros_like(pres_ref)

    enc = enc_ref[...]
    Ls = [jnp.dot(enc, w[0], preferred_element_type=jnp.float32)
          for w in (w0_ref, w1_ref, w2_ref, w3_ref)]
    iota = jax.lax.broadcasted_iota(jnp.int32, (RB, EC), 1)
    tgt = tgt_ref[0]
    vals = vals_ref[0]
    v = valid_ref[...]

    m = jnp.max(Ls[0], axis=1, keepdims=True)
    for L in Ls[1:]:
        m = jnp.maximum(m, jnp.max(L, axis=1, keepdims=True))
    se = jnp.sum(jnp.exp(Ls[0] - m), axis=1, keepdims=True)
    for L in Ls[1:]:
        se = se + jnp.sum(jnp.exp(L - m), axis=1, keepdims=True)
    lse = m + jnp.log(se)

    tl = jnp.sum(jnp.where(iota == tgt, Ls[0], 0.0), axis=1, keepdims=True)
    for h, L in enumerate(Ls[1:], start=1):
        tl = tl + jnp.sum(jnp.where(iota == (tgt - h * EC), L, 0.0),
                          axis=1, keepdims=True)

    nll_ref[...] = nll_ref[...] + jnp.sum(v * (lse - tl))
    corr_ref[...] = corr_ref[...] + jnp.sum(
        v * (tl == m).astype(jnp.float32))

    for h, L in enumerate(Ls):
        pres_ref[h:h + 1, :] = pres_ref[h:h + 1, :] + jnp.sum(
            (iota == (vals - h * EC)).astype(jnp.float32),
            axis=0, keepdims=True)

    @pl.when(jnp.logical_and(cb == NCB - 1, r == NRB - 1))
    def _():
        uniq_ref[...] = jnp.zeros_like(uniq_ref) + jnp.sum(
            (pres_ref[...] > 0).astype(jnp.float32))
        denom = msum_ref[0, 0] * NCB
        nll_ref[...] = nll_ref[...] / denom
        corr_ref[...] = corr_ref[...] / denom


def kernel(feats, feats_lengths, projection, embeddings, W_in, b_in,
           W_ff1, b_ff1, W_ff2, b_ff2, top_n_out):
    x = feats.reshape(ROWS, IN_DIM)
    embT = jnp.transpose(embeddings, (1, 2, 0))

    FH = 2 * D_MODEL
    enc, xsn = pl.pallas_call(
        _prep_kernel,
        grid=(1,),
        in_specs=[
            pl.BlockSpec((ROWS, IN_DIM), lambda i: (0, 0)),
            pl.BlockSpec((IN_DIM, NCB * EMB_DIM), lambda i: (0, 0)),
            pl.BlockSpec((IN_DIM, D_MODEL), lambda i: (0, 0)),
            pl.BlockSpec((1, D_MODEL), lambda i: (0, 0)),
            pl.BlockSpec((D_MODEL, FH), lambda i: (0, 0)),
            pl.BlockSpec((D_MODEL, FH), lambda i: (0, 1)),
            pl.BlockSpec((1, 4 * D_MODEL), lambda i: (0, 0)),
            pl.BlockSpec((FH, D_MODEL), lambda i: (0, 0)),
            pl.BlockSpec((FH, D_MODEL), lambda i: (1, 0)),
            pl.BlockSpec((1, D_MODEL), lambda i: (0, 0)),
        ],
        out_specs=[
            pl.BlockSpec((ROWS, D_MODEL), lambda i: (0, 0)),
            pl.BlockSpec((ROWS, NCB * EMB_DIM), lambda i: (0, 0)),
        ],
        out_shape=[jax.ShapeDtypeStruct((ROWS, D_MODEL), jnp.float32),
                   jax.ShapeDtypeStruct((ROWS, NCB * EMB_DIM), jnp.float32)],
    )(x, projection, W_in, b_in.reshape(1, -1),
      W_ff1, W_ff1, b_ff1.reshape(1, -1), W_ff2, W_ff2,
      b_ff2.reshape(1, -1))

    xsn3 = jnp.transpose(xsn.reshape(ROWS, NCB, EMB_DIM), (1, 0, 2))

    lim = feats_lengths // STRIDE
    t_idx = jnp.arange(N)
    validf = ((t_idx[None, :] + 1) < lim[:, None]).astype(jnp.float32) \
        .reshape(ROWS, 1)
    tmask_i = (t_idx[None, :] < lim[:, None]).astype(jnp.int32) \
        .reshape(ROWS, 1)
    t0_i = (jnp.arange(ROWS) % N == 0).astype(jnp.int32).reshape(ROWS, 1)
    mask_sum = jnp.sum(jnp.maximum(lim - 1, 0).astype(jnp.float32))
    msum = mask_sum.reshape(1, 1)

    tgt, vals2 = pl.pallas_call(
        _codes_kernel,
        grid=(NCB, NE),
        in_specs=[
            pl.BlockSpec((1, ROWS, EMB_DIM), lambda cb, ec: (cb, 0, 0)),
            pl.BlockSpec((1, EMB_DIM, EC), lambda cb, ec: (cb, 0, ec)),
            pl.BlockSpec((ROWS, 1), lambda cb, ec: (0, 0)),
            pl.BlockSpec((ROWS, 1), lambda cb, ec: (0, 0)),
        ],
        out_specs=[pl.BlockSpec((1, ROWS, 1), lambda cb, ec: (cb, 0, 0))] * 2,
        out_shape=[jax.ShapeDtypeStruct((NCB, ROWS, 1), jnp.int32)] * 2,
        scratch_shapes=[pltpu.VMEM((ROWS, 1), jnp.float32),
                        pltpu.VMEM((ROWS, 1), jnp.int32)],
    )(xsn3, embT, tmask_i, t0_i)

    W = top_n_out[0]

    def _wspec(h):
        return pl.BlockSpec((1, D_MODEL, EC), lambda cb, r: (cb, 0, h))

    lossv, acc, uniq = pl.pallas_call(
        _loss_kernel,
        grid=(NCB, NRB),
        in_specs=[
            pl.BlockSpec((RB, D_MODEL), lambda cb, r: (r, 0)),
            _wspec(0), _wspec(1), _wspec(2), _wspec(3),
            pl.BlockSpec((1, RB, 1), lambda cb, r: (cb, r, 0)),
            pl.BlockSpec((1, RB, 1), lambda cb, r: (cb, r, 0)),
            pl.BlockSpec((RB, 1), lambda cb, r: (r, 0)),
            pl.BlockSpec((1, 1), lambda cb, r: (0, 0)),
        ],
        out_specs=[pl.BlockSpec((1, 1), lambda cb, r: (0, 0))] * 3,
        out_shape=[jax.ShapeDtypeStruct((1, 1), jnp.float32)] * 3,
        scratch_shapes=[pltpu.VMEM((NE, EC), jnp.float32)],
    )(enc, W, W, W, W, tgt, vals2, validf, msum)

    num_codes = mask_sum * NCB
    return (acc[0, 0], lossv[0, 0], num_codes, uniq[0, 0].astype(jnp.int32))


# speedup vs baseline: 1.2887x; 1.0239x over previous
import jax
import jax.numpy as jnp
from jax.experimental import pallas as pl
from jax.experimental.pallas import tpu as pltpu

B, T, NMEL = 4, 600, 80
STRIDE = 4
IN_DIM = NMEL * STRIDE
D_MODEL = 512
NCB = 4
EMB_DIM = 16
NUM_EMB = 8192
N = T // STRIDE
ROWS = B * N
EC = 2048
NE = NUM_EMB // EC
NE2 = NE // 2
RB = 200
NRB = ROWS // RB
NEG = -1e30


def _ln(x, eps=1e-6):
    m = jnp.mean(x, axis=-1, keepdims=True)
    s = x - m
    v = jnp.mean(s * s, axis=-1, keepdims=True)
    return s / jnp.sqrt(v + eps)


def _prep_kernel(x_ref, proj_ref, w_in_ref, b_in_ref,
                 w_ff1a_ref, w_ff1b_ref, b_ff1_ref,
                 w_ff2a_ref, w_ff2b_ref, b_ff2_ref,
                 enc_ref, xsn_ref):
    x = x_ref[...]
    y = _ln(x)
    xs = jnp.dot(y, proj_ref[...], preferred_element_type=jnp.float32)
    nrm = jnp.sqrt(jnp.sum(xs * xs, axis=-1, keepdims=True))
    xsn_ref[...] = xs / (nrm + 1e-8)
    h1 = jnp.dot(x, w_in_ref[...], preferred_element_type=jnp.float32) \
        + b_in_ref[...]
    t = _ln(h1)
    FH = 2 * D_MODEL
    b_ff1 = b_ff1_ref[...]
    f0 = jax.nn.gelu(
        jnp.dot(t, w_ff1a_ref[...], preferred_element_type=jnp.float32)
        + b_ff1[:, :FH])
    f1 = jax.nn.gelu(
        jnp.dot(t, w_ff1b_ref[...], preferred_element_type=jnp.float32)
        + b_ff1[:, FH:])
    h2 = h1 \
        + jnp.dot(f0, w_ff2a_ref[...], preferred_element_type=jnp.float32) \
        + jnp.dot(f1, w_ff2b_ref[...], preferred_element_type=jnp.float32) \
        + b_ff2_ref[...]
    enc_ref[...] = _ln(h2)


def _codes_kernel(xsn_ref, embt_ref, tmask_ref, t0_ref,
                  tgt_ref, vals_ref, runmin_ref, runidx_ref):
    ec = pl.program_id(0)

    @pl.when(ec == 0)
    def _():
        runmin_ref[...] = jnp.full_like(runmin_ref, jnp.inf)
        runidx_ref[...] = jnp.full_like(runidx_ref, NUM_EMB)

    iota = jax.lax.broadcasted_iota(jnp.int32, (ROWS, EC), 1)
    for cb in range(NCB):
        xq = xsn_ref[cb]
        et = embt_ref[cb]
        c2 = jnp.sum(et * et, axis=0, keepdims=True)
        d = c2 - 2.0 * jnp.dot(xq, et, preferred_element_type=jnp.float32)
        m = jnp.min(d, axis=1, keepdims=True)
        idx = jnp.min(jnp.where(d == m, iota, NUM_EMB), axis=1,
                      keepdims=True) + ec * EC
        col = pl.ds(cb, 1)
        upd = m < runmin_ref[:, col]
        runidx_ref[:, col] = jnp.where(upd, idx, runidx_ref[:, col])
        runmin_ref[:, col] = jnp.where(upd, m, runmin_ref[:, col])

    @pl.when(ec == NE - 1)
    def _():
        t0 = t0_ref[...]
        tmask = tmask_ref[...]
        for cb in range(NCB):
            codes = runidx_ref[:, pl.ds(cb, 1)]
            tgt_ref[cb] = jnp.concatenate([codes[1:, :], codes[:1, :]],
                                          axis=0)
            vals_ref[cb] = jnp.where(t0 != 0, -1,
                                     jnp.where(tmask != 0, codes, 0))


def _loss_kernel(enc_ref, w0_ref, w1_ref, w2_ref, w3_ref,
                 tgt_ref, vals_ref, valid_ref, msum_ref,
                 nll_ref, corr_ref, uniq_ref, pres_ref):
    cb = pl.program_id(0)
    r = pl.program_id(1)

    @pl.when(jnp.logical_and(cb == 0, r == 0))
    def _():
        nll_ref[...] = jnp.zeros_like(nll_ref)
        corr_ref[...] = jnp.zeros_like(corr_ref)
        uniq_ref[...] = jnp.zeros_like(uniq_ref)
        pres_ref[...] = jnp.zeros_like(pres_ref)

    enc = enc_ref[...]
    Ls = [jnp.dot(enc, w[0], preferred_element_type=jnp.float32)
          for w in (w0_ref, w1_ref, w2_ref, w3_ref)]
    iota = jax.lax.broadcasted_iota(jnp.int32, (RB, EC), 1)
    tgt = tgt_ref[0]
    vals = vals_ref[0]
    v = valid_ref[...]

    m = jnp.max(Ls[0], axis=1, keepdims=True)
    for L in Ls[1:]:
        m = jnp.maximum(m, jnp.max(L, axis=1, keepdims=True))
    se = jnp.sum(jnp.exp(Ls[0] - m), axis=1, keepdims=True)
    for L in Ls[1:]:
        se = se + jnp.sum(jnp.exp(L - m), axis=1, keepdims=True)
    lse = m + jnp.log(se)

    tl = jnp.sum(jnp.where(iota == tgt, Ls[0], 0.0), axis=1, keepdims=True)
    for h, L in enumerate(Ls[1:], start=1):
        tl = tl + jnp.sum(jnp.where(iota == (tgt - h * EC), L, 0.0),
                          axis=1, keepdims=True)

    nll_ref[...] = nll_ref[...] + jnp.sum(v * (lse - tl))
    corr_ref[...] = corr_ref[...] + jnp.sum(
        v * (tl == m).astype(jnp.float32))

    for h, L in enumerate(Ls):
        pres_ref[h:h + 1, :] = pres_ref[h:h + 1, :] + jnp.sum(
            (iota == (vals - h * EC)).astype(jnp.float32),
            axis=0, keepdims=True)

    @pl.when(jnp.logical_and(cb == NCB - 1, r == NRB - 1))
    def _():
        uniq_ref[...] = jnp.zeros_like(uniq_ref) + jnp.sum(
            (pres_ref[...] > 0).astype(jnp.float32))
        denom = msum_ref[0, 0] * NCB
        nll_ref[...] = nll_ref[...] / denom
        corr_ref[...] = corr_ref[...] / denom


def kernel(feats, feats_lengths, projection, embeddings, W_in, b_in,
           W_ff1, b_ff1, W_ff2, b_ff2, top_n_out):
    x = feats.reshape(ROWS, IN_DIM)
    embT = jnp.transpose(embeddings, (1, 2, 0))

    FH = 2 * D_MODEL
    enc, xsn = pl.pallas_call(
        _prep_kernel,
        grid=(1,),
        in_specs=[
            pl.BlockSpec((ROWS, IN_DIM), lambda i: (0, 0)),
            pl.BlockSpec((IN_DIM, NCB * EMB_DIM), lambda i: (0, 0)),
            pl.BlockSpec((IN_DIM, D_MODEL), lambda i: (0, 0)),
            pl.BlockSpec((1, D_MODEL), lambda i: (0, 0)),
            pl.BlockSpec((D_MODEL, FH), lambda i: (0, 0)),
            pl.BlockSpec((D_MODEL, FH), lambda i: (0, 1)),
            pl.BlockSpec((1, 4 * D_MODEL), lambda i: (0, 0)),
            pl.BlockSpec((FH, D_MODEL), lambda i: (0, 0)),
            pl.BlockSpec((FH, D_MODEL), lambda i: (1, 0)),
            pl.BlockSpec((1, D_MODEL), lambda i: (0, 0)),
        ],
        out_specs=[
            pl.BlockSpec((ROWS, D_MODEL), lambda i: (0, 0)),
            pl.BlockSpec((ROWS, NCB * EMB_DIM), lambda i: (0, 0)),
        ],
        out_shape=[jax.ShapeDtypeStruct((ROWS, D_MODEL), jnp.float32),
                   jax.ShapeDtypeStruct((ROWS, NCB * EMB_DIM), jnp.float32)],
    )(x, projection, W_in, b_in.reshape(1, -1),
      W_ff1, W_ff1, b_ff1.reshape(1, -1), W_ff2, W_ff2,
      b_ff2.reshape(1, -1))

    xsn3 = jnp.transpose(xsn.reshape(ROWS, NCB, EMB_DIM), (1, 0, 2))

    lim = feats_lengths // STRIDE
    t_idx = jnp.arange(N)
    validf = ((t_idx[None, :] + 1) < lim[:, None]).astype(jnp.float32) \
        .reshape(ROWS, 1)
    tmask_i = (t_idx[None, :] < lim[:, None]).astype(jnp.int32) \
        .reshape(ROWS, 1)
    t0_i = (jnp.arange(ROWS) % N == 0).astype(jnp.int32).reshape(ROWS, 1)
    mask_sum = jnp.sum(jnp.maximum(lim - 1, 0).astype(jnp.float32))
    msum = mask_sum.reshape(1, 1)

    tgt, vals2 = pl.pallas_call(
        _codes_kernel,
        grid=(NE,),
        in_specs=[
            pl.BlockSpec((NCB, ROWS, EMB_DIM), lambda ec: (0, 0, 0)),
            pl.BlockSpec((NCB, EMB_DIM, EC), lambda ec: (0, 0, ec)),
            pl.BlockSpec((ROWS, 1), lambda ec: (0, 0)),
            pl.BlockSpec((ROWS, 1), lambda ec: (0, 0)),
        ],
        out_specs=[pl.BlockSpec((NCB, ROWS, 1), lambda ec: (0, 0, 0))] * 2,
        out_shape=[jax.ShapeDtypeStruct((NCB, ROWS, 1), jnp.int32)] * 2,
        scratch_shapes=[pltpu.VMEM((ROWS, NCB), jnp.float32),
                        pltpu.VMEM((ROWS, NCB), jnp.int32)],
    )(xsn3, embT, tmask_i, t0_i)

    W = top_n_out[0]

    def _wspec(h):
        return pl.BlockSpec((1, D_MODEL, EC), lambda cb, r: (cb, 0, h))

    lossv, acc, uniq = pl.pallas_call(
        _loss_kernel,
        grid=(NCB, NRB),
        in_specs=[
            pl.BlockSpec((RB, D_MODEL), lambda cb, r: (r, 0)),
            _wspec(0), _wspec(1), _wspec(2), _wspec(3),
            pl.BlockSpec((1, RB, 1), lambda cb, r: (cb, r, 0)),
            pl.BlockSpec((1, RB, 1), lambda cb, r: (cb, r, 0)),
            pl.BlockSpec((RB, 1), lambda cb, r: (r, 0)),
            pl.BlockSpec((1, 1), lambda cb, r: (0, 0)),
        ],
        out_specs=[pl.BlockSpec((1, 1), lambda cb, r: (0, 0))] * 3,
        out_shape=[jax.ShapeDtypeStruct((1, 1), jnp.float32)] * 3,
        scratch_shapes=[pltpu.VMEM((NE, EC), jnp.float32)],
    )(enc, W, W, W, W, tgt, vals2, validf, msum)

    num_codes = mask_sum * NCB
    return (acc[0, 0], lossv[0, 0], num_codes, uniq[0, 0].astype(jnp.int32))
